# baseline (speedup 1.0000x reference)
import math

import jax
import jax.numpy as jnp
import numpy as np
from jax import lax
from jax.experimental import pallas as pl
from jax.experimental.pallas import tpu as pltpu
from jax.experimental.pallas import tpu_sc as plsc

_D_MODEL = 128
_SEQ = 200
_BATCH = 1024
_SCALE = math.sqrt(float(_D_MODEL))

_NUM_CORES = 2
_NUM_SUBCORES = 16
_NW = _NUM_CORES * _NUM_SUBCORES
_SEQS_PER_W = _BATCH // _NW
_VREGS_PER_ROW = _D_MODEL // 16


def _make_pe():
    pe = np.zeros((_SEQ, _D_MODEL), dtype=np.float32)
    position = np.arange(0, _SEQ, dtype=np.float32)[:, None]
    div_term = np.exp(
        np.arange(0, _D_MODEL, 2, dtype=np.float32)
        * -(math.log(10000.0) / _D_MODEL)
    )
    pe[:, 0::2] = np.sin(position * div_term)
    pe[:, 1::2] = np.cos(position * div_term)
    return (pe.astype(np.float64) / math.sqrt(float(_D_MODEL))).astype(
        np.float32)


_PE = _make_pe()


_NBUF = 4
_NGROUPS = _SEQS_PER_W // _NBUF


def _body(lut_hbm, idx_hbm, pe_hbm, out_hbm,
          idx0, idx1, idx2, idx3, rows0, rows1, rows2, rows3,
          isem0, isem1, isem2, isem3, gsem0, gsem1, gsem2, gsem3,
          ssem0, ssem1, ssem2, ssem3, psem0, psem1, psem2, psem3):
    idxb = (idx0, idx1, idx2, idx3)
    rows = (rows0, rows1, rows2, rows3)
    isem = (isem0, isem1, isem2, isem3)
    gsem = (gsem0, gsem1, gsem2, gsem3)
    ssem = (ssem0, ssem1, ssem2, ssem3)
    psem = (psem0, psem1, psem2, psem3)
    wid = lax.axis_index("s") * _NUM_CORES + lax.axis_index("c")
    wbase = wid * _SEQS_PER_W

    def fire_prefill(p):
        pltpu.async_copy(pe_hbm, rows[p], psem[p])

    def wait_prefill(p):
        pltpu.make_async_copy(pe_hbm, rows[p], psem[p]).wait()

    def fire_idx(s, p):
        pltpu.async_copy(
            idx_hbm.at[pl.ds((wbase + s) * _SEQ, _SEQ)], idxb[p], isem[p])

    def wait_idx(p):
        pltpu.make_async_copy(
            idx_hbm.at[pl.ds(0, _SEQ)], idxb[p], isem[p]).wait()

    def fire_gather(p):
        pltpu.async_copy(lut_hbm.at[idxb[p]], rows[p], gsem[p], add=True)

    def wait_gather(p):
        pltpu.make_async_copy(lut_hbm.at[idxb[p]], rows[p], gsem[p]).wait()

    def fire_store(s, p):
        pltpu.async_copy(
            rows[p], out_hbm.at[pl.ds((wbase + s) * _SEQ, _SEQ)], ssem[p])

    def wait_store(p):
        pltpu.make_async_copy(
            rows[p], out_hbm.at[pl.ds(0, _SEQ)], ssem[p]).wait()

    def compute(p):
        @plsc.parallel_loop(0, _SEQ, unroll=4)
        def _row_loop(r):
            for j in range(_VREGS_PER_ROW):
                sl = pl.ds(j * 16, 16)
                rows[p][r, sl] = rows[p][r, sl] * _SCALE

    fire_idx(0, 0)
    fire_idx(1, 1)
    fire_idx(2, 2)
    fire_prefill(0)
    fire_prefill(1)
    fire_prefill(2)
    wait_idx(0)
    wait_prefill(0)
    fire_gather(0)
    wait_idx(1)
    wait_prefill(1)
    fire_gather(1)

    @pl.loop(0, _NGROUPS)
    def _group(g):
        for b in range(_NBUF):
            s = 4 * g + b
            q = (b + 2) % _NBUF
            p3 = (b + 3) % _NBUF
            wait_gather(b)
            compute(b)
            fire_store(s, b)
            if b == 0:
                fire_idx(s + 3, p3)

                @pl.when(g >= 1)
                def _():
                    wait_store(p3)
                fire_prefill(p3)
            else:
                wait_store(p3)

                @pl.when(g < _NGROUPS - 1)
                def _():
                    fire_idx(s + 3, p3)
                    fire_prefill(p3)
            if b < 2:
                wait_idx(q)
                wait_prefill(q)
                fire_gather(q)
            else:
                @pl.when(g < _NGROUPS - 1)
                def _():
                    wait_idx(q)
                    wait_prefill(q)
                    fire_gather(q)

    wait_store(3)


@jax.jit
def _run(lut, idx, pe):
    kern = pl.kernel(
        _body,
        out_type=jax.ShapeDtypeStruct((_BATCH * _SEQ, _D_MODEL), jnp.float32),
        mesh=plsc.VectorSubcoreMesh(
            core_axis_name="c", subcore_axis_name="s",
            num_cores=_NUM_CORES, num_subcores=_NUM_SUBCORES,
        ),
        scratch_types=(
            [pltpu.VMEM((_SEQ,), jnp.int32)] * _NBUF
            + [pltpu.VMEM((_SEQ, _D_MODEL), jnp.float32)] * _NBUF
            + [pltpu.SemaphoreType.DMA] * (4 * _NBUF)
        ),
    )
    return kern(lut, idx, pe)


def kernel(x, lut):
    idx = x.reshape(-1).astype(jnp.int32)
    pe = jnp.asarray(_PE)
    return _run(lut, idx, pe).reshape(_BATCH, _SEQ, _D_MODEL)


# speedup vs baseline: 2.2691x; 2.2691x over previous
import math

import jax
import jax.numpy as jnp
import numpy as np
from jax import lax
from jax.experimental import pallas as pl
from jax.experimental.pallas import tpu as pltpu
from jax.experimental.pallas import tpu_sc as plsc

_D_MODEL = 128
_SEQ = 200
_BATCH = 1024
_SCALE = math.sqrt(float(_D_MODEL))

_NUM_CORES = 2
_NUM_SUBCORES = 16
_NW = _NUM_CORES * _NUM_SUBCORES
_SEQS_PER_W = _BATCH // _NW
_VREGS_PER_ROW = _D_MODEL // 16

_QUAD = 4
_NCHUNK = 5
_CHUNK = _SEQ // _NCHUNK
_NBANK = 5
_KGROUPS = _SEQS_PER_W // _QUAD


def _make_pe():
    pe = np.zeros((_SEQ, _D_MODEL), dtype=np.float32)
    position = np.arange(0, _SEQ, dtype=np.float32)[:, None]
    div_term = np.exp(
        np.arange(0, _D_MODEL, 2, dtype=np.float32)
        * -(math.log(10000.0) / _D_MODEL)
    )
    pe[:, 0::2] = np.sin(position * div_term)
    pe[:, 1::2] = np.cos(position * div_term)
    return pe


_PE = _make_pe()


def _body(lut_hbm, idx_hbm, pe_hbm, out_hbm, *scr):
    nb = _NBANK * _QUAD
    idxb = scr[0:nb]
    rows = scr[nb:2 * nb]
    pe_v = scr[2 * nb]
    isem = scr[2 * nb + 1:2 * nb + 1 + _NBANK]
    gsem = scr[2 * nb + 1 + _NBANK:2 * nb + 1 + 2 * _NBANK]
    ssem = scr[2 * nb + 1 + 2 * _NBANK:2 * nb + 1 + 3 * _NBANK]
    wid = lax.axis_index("s") * _NUM_CORES + lax.axis_index("c")
    wbase = wid * _SEQS_PER_W
    pltpu.sync_copy(pe_hbm, pe_v)

    def chunk_base(k, i, j):
        return (wbase + _QUAD * k + i) * _SEQ + _CHUNK * j

    def fire_idxs(k, j, a):
        for i in range(_QUAD):
            pltpu.async_copy(
                idx_hbm.at[pl.ds(chunk_base(k, i, j), _CHUNK)],
                idxb[_QUAD * a + i], isem[a])

    def wait_idxs(a):
        for i in range(_QUAD):
            pltpu.make_async_copy(
                idx_hbm.at[pl.ds(0, _CHUNK)],
                idxb[_QUAD * a + i], isem[a]).wait()

    def fire_gathers(a):
        for i in range(_QUAD):
            pltpu.async_copy(
                lut_hbm.at[idxb[_QUAD * a + i]], rows[_QUAD * a + i], gsem[a])

    def wait_gathers(a):
        for i in range(_QUAD):
            pltpu.make_async_copy(
                lut_hbm.at[idxb[_QUAD * a + i]], rows[_QUAD * a + i],
                gsem[a]).wait()

    def fire_stores(k, j, a):
        for i in range(_QUAD):
            pltpu.async_copy(
                rows[_QUAD * a + i],
                out_hbm.at[pl.ds(chunk_base(k, i, j), _CHUNK)], ssem[a])

    def wait_stores(a):
        for i in range(_QUAD):
            pltpu.make_async_copy(
                rows[_QUAD * a + i],
                out_hbm.at[pl.ds(0, _CHUNK)], ssem[a]).wait()

    def compute_quad(a, j):
        rbs = [rows[_QUAD * a + i] for i in range(_QUAD)]

        @plsc.parallel_loop(0, _CHUNK, unroll=2)
        def _row_loop(r):
            for jj in range(_VREGS_PER_ROW):
                sl = pl.ds(jj * 16, 16)
                pe_reg = pe_v[_CHUNK * j + r, sl]
                for rb in rbs:
                    rb[r, sl] = rb[r, sl] * _SCALE + pe_reg

    fire_idxs(0, 0, 0)
    fire_idxs(0, 1, 1)
    fire_idxs(0, 2, 2)
    wait_idxs(0)
    fire_gathers(0)
    wait_idxs(1)
    fire_gathers(1)

    @pl.loop(0, _KGROUPS)
    def _group(k):
        for j in range(_NCHUNK):
            a = j
            wait_gathers(a)
            compute_quad(a, j)
            fire_stores(k, j, a)
            i3 = (j + 3) % _NCHUNK
            if j <= 1:
                fire_idxs(k, j + 3, i3)
            else:
                @pl.when(k < _KGROUPS - 1)
                def _():
                    fire_idxs(k + 1, (j + 3) % _NCHUNK, i3)
            g2 = (j + 2) % _NCHUNK
            if j <= 2:
                @pl.when(k >= 1)
                def _():
                    wait_stores(g2)
                wait_idxs(g2)
                fire_gathers(g2)
            else:
                wait_stores(g2)

                @pl.when(k < _KGROUPS - 1)
                def _():
                    wait_idxs(g2)
                    fire_gathers(g2)

    wait_stores(2)
    wait_stores(3)
    wait_stores(4)


@jax.jit
def _run(lut, idx, pe):
    nb = _NBANK * _QUAD
    kern = pl.kernel(
        _body,
        out_type=jax.ShapeDtypeStruct((_BATCH * _SEQ, _D_MODEL), jnp.float32),
        mesh=plsc.VectorSubcoreMesh(
            core_axis_name="c", subcore_axis_name="s",
            num_cores=_NUM_CORES, num_subcores=_NUM_SUBCORES,
        ),
        scratch_types=(
            [pltpu.VMEM((_CHUNK,), jnp.int32)] * nb
            + [pltpu.VMEM((_CHUNK, _D_MODEL), jnp.float32)] * nb
            + [pltpu.VMEM((_SEQ, _D_MODEL), jnp.float32)]
            + [pltpu.SemaphoreType.DMA] * (3 * _NBANK)
        ),
    )
    return kern(lut, idx, pe)


def kernel(x, lut):
    idx = x.reshape(-1).astype(jnp.int32)
    pe = jnp.asarray(_PE)
    return _run(lut, idx, pe).reshape(_BATCH, _SEQ, _D_MODEL)
